# baseline (speedup 1.0000x reference)
import functools

import jax
import jax.numpy as jnp
from jax import lax
from jax.experimental import pallas as pl
from jax.experimental.pallas import tpu as pltpu
from jax.experimental.pallas import tpu_sc as plsc

_B, _H, _W = 64, 48, 48
_HW = _H * _W
_R, _L = 18, 128
_MAX_GROUP = 1000
_PADG = 1024
_BPP = 4
_GRID = _B // _BPP

_NC, _NS = 2, 16
_NW = _NC * _NS
_BPW = _B // _NW
_CHUNKS = _HW // 16


def _leaky(x):
    return jnp.where(x >= 0, x, 0.01 * x)


def _fill(buf, x):
    buf[pl.ds(1, _R), pl.ds(0, _L)] = x
    buf[pl.ds(0, _R), pl.ds(_L, _L)] = x


def _shift(buf, s):
    a, r = (-1, s + _L) if s < 0 else (0, s)
    return buf[pl.ds(1 + a, _R), pl.ds(r, _L)]


def _shift2d(buf, dy, dx, colmasks):
    s = _shift(buf, dy * _W + dx)
    return s * colmasks[dx] if dx else s


def _avg3(buf, x, colmasks):
    _fill(buf, x)
    acc = None
    for dy in (-1, 0, 1):
        for dx in (-1, 0, 1):
            v = _shift2d(buf, dy, dx, colmasks)
            acc = v if acc is None else acc + v
    return acc * (1.0 / 9.0)


def _dense_body(gf_ref, map_ref, fac_ref, unit_ref, loc_ref,
                va_fact_ref, va_move_ref, va_transfer_ref, va_pickup_ref,
                va_dig_ref, va_sd_ref, va_rech_ref, va_dn_ref,
                g_W, g_b, f_W, f_b, u_W, u_b, m_W, m_b, ld_W, ld_b, c_W, c_b,
                cvu_ref, cvf_ref, idsu_ref, idsf_ref, buf):
    g = pl.program_id(0)

    @pl.when(g == 0)
    def _():
        buf[...] = jnp.zeros((24, 2 * _L), jnp.float32)

    bi = lax.broadcasted_iota(jnp.int32, (_R, _L), 0)
    li = lax.broadcasted_iota(jnp.int32, (_R, _L), 1)
    w = jnp.remainder(bi * _L + li, _W)
    colmasks = {}
    for dx in (-2, -1, 1, 2):
        wd = w + dx
        colmasks[dx] = jnp.where((wd >= 0) & (wd < _W), 1.0, 0.0)

    def img(ref, j, c, nch):
        return ref[pl.ds((j * nch + c) * _R, _R), :]

    for j in range(_BPP):
        gf0 = gf_ref[g * _BPP + j, 0]
        gf1 = gf_ref[g * _BPP + j, 1]
        sg = c_b[0]
        for o in range(2):
            ge = _leaky(g_W[o, 0] * gf0 + g_W[o, 1] * gf1 + g_b[o])
            sg = sg + c_W[0, o] * ge

        def conv1x1(ref, wm, bm, o, nch):
            acc = None
            for c in range(nch):
                v = img(ref, j, c, nch) * wm[o, c]
                acc = v if acc is None else acc + v
            return _leaky(acc + bm[o])

        me = [conv1x1(map_ref, m_W, m_b, o, 6) for o in range(2)]

        q0 = _avg3(buf, me[0], colmasks)
        q1 = _avg3(buf, me[1], colmasks)
        z = [None] * 8
        for i, q in enumerate((q0, q1)):
            _fill(buf, q)
            for dy in range(-2, 3):
                for dx in range(-2, 3):
                    s = _shift2d(buf, dy, dx, colmasks)
                    for o in range(8):
                        wgt = ld_W[o, i * 25 + (dy + 2) * 5 + (dx + 2)]
                        z[o] = s * wgt if z[o] is None else z[o] + s * wgt
        t = None
        for o in range(8):
            u = _leaky(z[o] + ld_b[o])
            v = c_W[0, 8 + o] * u
            t = v if t is None else t + v
        crit = _avg3(buf, t, colmasks) + sg

        for o in range(2):
            crit = crit + c_W[0, 2 + o] * conv1x1(fac_ref, f_W, f_b, o, 6)
            crit = crit + c_W[0, 4 + o] * conv1x1(unit_ref, u_W, u_b, o, 4)
            crit = crit + c_W[0, 6 + o] * me[o]

        def any_over(ref, nch):
            acc = img(ref, j, 0, nch)
            for k in range(1, nch):
                acc = jnp.logical_or(acc, img(ref, j, k, nch))
            return acc

        fm = any_over(va_fact_ref, 4)
        um = any_over(va_dn_ref, 1)
        for ref, nch in ((va_move_ref, 10), (va_transfer_ref, 50),
                         (va_pickup_ref, 10), (va_dig_ref, 2),
                         (va_sd_ref, 2), (va_rech_ref, 2)):
            um = jnp.logical_or(um, any_over(ref, nch))

        orow = pl.ds(j * _R, _R)
        cvu_ref[orow, :] = jnp.where(um, crit, 0.0)
        cvf_ref[orow, :] = jnp.where(fm, crit, 0.0)
        idsf_ref[orow, :] = img(loc_ref, j, 0, 2).astype(jnp.int32)
        idsu_ref[orow, :] = img(loc_ref, j, 1, 2).astype(jnp.int32)


def _tc_dense(global_feature, map_feature, factory_feature, unit_feature,
              location_feature,
              va_fact, va_move, va_transfer, va_pickup, va_dig, va_sd,
              va_rech, va_dn,
              g_W, g_b, f_W, f_b, u_W, u_b, m_W, m_b, ld_Wr, ld_b, c_W, c_b):
    def row_spec(nch):
        return pl.BlockSpec((_BPP * nch * _R, _L), lambda g: (g, 0))

    smem = pl.BlockSpec(memory_space=pltpu.SMEM)
    in_specs = [
        smem,
        row_spec(6), row_spec(6), row_spec(4), row_spec(2),
        row_spec(4), row_spec(10), row_spec(50), row_spec(10),
        row_spec(2), row_spec(2), row_spec(2), row_spec(1),
    ] + [smem] * 12
    out_spec = pl.BlockSpec((_BPP * _R, _L), lambda g: (g, 0))
    return pl.pallas_call(
        _dense_body,
        grid=(_GRID,),
        in_specs=in_specs,
        out_specs=[out_spec] * 4,
        out_shape=[jax.ShapeDtypeStruct((_B * _R, _L), jnp.float32)] * 2
        + [jax.ShapeDtypeStruct((_B * _R, _L), jnp.int32)] * 2,
        scratch_shapes=[pltpu.VMEM((24, 2 * _L), jnp.float32)],
    )(global_feature, map_feature, factory_feature, unit_feature,
      location_feature,
      va_fact, va_move, va_transfer, va_pickup, va_dig, va_sd, va_rech, va_dn,
      g_W, g_b, f_W, f_b, u_W, u_b, m_W, m_b, ld_Wr, ld_b, c_W, c_b)


def _sc_scatter(ids_u, ids_f, cv_u, cv_f):
    mesh = plsc.VectorSubcoreMesh(core_axis_name="c", subcore_axis_name="s")

    @functools.partial(
        pl.kernel,
        out_type=jax.ShapeDtypeStruct((_B * _PADG,), jnp.float32),
        mesh=mesh,
        scratch_types=[
            pltpu.VMEM((_HW,), jnp.int32),
            pltpu.VMEM((_HW,), jnp.int32),
            pltpu.VMEM((_HW,), jnp.float32),
            pltpu.VMEM((_HW,), jnp.float32),
            pltpu.VMEM((_PADG,), jnp.float32),
        ],
        compiler_params=pltpu.CompilerParams(needs_layout_passes=False),
    )
    def run(idsu_hbm, idsf_hbm, cvu_hbm, cvf_hbm, out_hbm,
            idsu_v, idsf_v, cvu_v, cvf_v, acc_v):
        wid = lax.axis_index("s") * _NC + lax.axis_index("c")
        for j in range(_BPW):
            b = wid * _BPW + j
            row = pl.ds(b * _HW, _HW)
            pltpu.sync_copy(idsu_hbm.at[row], idsu_v)
            pltpu.sync_copy(idsf_hbm.at[row], idsf_v)
            pltpu.sync_copy(cvu_hbm.at[row], cvu_v)
            pltpu.sync_copy(cvf_hbm.at[row], cvf_v)

            zeros16 = jnp.zeros((16,), jnp.float32)

            def zbody(i, _):
                acc_v[pl.ds(pl.multiple_of(i * 16, 16), 16)] = zeros16
                return 0

            lax.fori_loop(0, _PADG // 16, zbody, 0)

            def body(i, _):
                sl = pl.ds(pl.multiple_of(i * 16, 16), 16)
                plsc.addupdate_scatter(acc_v, [idsu_v[sl]], cvu_v[sl])
                plsc.addupdate_scatter(acc_v, [idsf_v[sl]], cvf_v[sl])
                return 0

            lax.fori_loop(0, _CHUNKS, body, 0)
            pltpu.sync_copy(acc_v, out_hbm.at[pl.ds(b * _PADG, _PADG)])

    return run(ids_u, ids_f, cv_u, cv_f)


def kernel(global_feature, map_feature, factory_feature, unit_feature,
           location_feature, va_factory_act, va_move, va_transfer, va_pickup,
           va_dig, va_self_destruct, va_recharge, va_do_nothing,
           g_W, g_b, f_W, f_b, u_W, u_b, m_W, m_b, ld_W, ld_b, c_W, c_b):
    Bn = global_feature.shape[0]
    cv_u, cv_f, ids_u, ids_f = _tc_dense(
        global_feature,
        map_feature.reshape(Bn * 6 * _R, _L),
        factory_feature.reshape(Bn * 6 * _R, _L),
        unit_feature.reshape(Bn * 4 * _R, _L),
        location_feature.reshape(Bn * 2 * _R, _L),
        va_factory_act.reshape(Bn * 4 * _R, _L),
        va_move.reshape(Bn * 10 * _R, _L),
        va_transfer.reshape(Bn * 50 * _R, _L),
        va_pickup.reshape(Bn * 10 * _R, _L),
        va_dig.reshape(Bn * 2 * _R, _L),
        va_self_destruct.reshape(Bn * 2 * _R, _L),
        va_recharge.reshape(Bn * 2 * _R, _L),
        va_do_nothing.reshape(Bn * 1 * _R, _L),
        g_W, g_b, f_W, f_b, u_W, u_b, m_W, m_b,
        ld_W.reshape(8, 50), ld_b, c_W, c_b)
    out = _sc_scatter(ids_u.reshape(-1), ids_f.reshape(-1),
                      cv_u.reshape(-1), cv_f.reshape(-1))
    return out.reshape(Bn, _PADG)[:, :_MAX_GROUP]


# speedup vs baseline: 2.9087x; 2.9087x over previous
import jax, jax.numpy as jnp
from sc_v3 import sc_scatter_v3

_B, _H, _W = 64, 48, 48
_MAX_GROUP = 1000


def _leaky(x):
    return jnp.where(x >= 0, x, 0.01 * x)


def _conv1x1(x, Wm, b):
    return jnp.einsum('bchw,oc->bohw', x, Wm) + b[None, :, None, None]


def _avg_pool3(x):
    s = jax.lax.reduce_window(x, 0.0, jax.lax.add, (1, 1, 3, 3), (1, 1, 1, 1), [(0, 0), (0, 0), (1, 1), (1, 1)])
    return s / 9.0


def _conv5_same(x, Wm, b):
    y = jax.lax.conv_general_dilated(x, Wm, (1, 1), 'SAME', dimension_numbers=('NCHW', 'OIHW', 'NCHW'))
    return y + b[None, :, None, None]


def kernel(global_feature, map_feature, factory_feature, unit_feature, location_feature, va_factory_act, va_move, va_transfer, va_pickup, va_dig, va_self_destruct, va_recharge, va_do_nothing, g_W, g_b, f_W, f_b, u_W, u_b, m_W, m_b, ld_W, ld_b, c_W, c_b):
    Bn = global_feature.shape[0]
    gf = jnp.broadcast_to(global_feature[:, :, None, None], (Bn, 2, _H, _W))
    ge = _leaky(_conv1x1(gf, g_W, g_b))
    fe = _leaky(_conv1x1(factory_feature, f_W, f_b))
    ue = _leaky(_conv1x1(unit_feature, u_W, u_b))
    me = _leaky(_conv1x1(map_feature, m_W, m_b))
    ld = _avg_pool3(_leaky(_conv5_same(_avg_pool3(me), ld_W, ld_b)))
    combined = jnp.concatenate([ge, fe, ue, me, ld], axis=1)
    uatv = jnp.stack([
        va_move.reshape(Bn, -1, _H, _W).any(axis=1),
        va_transfer.reshape(Bn, -1, _H, _W).any(axis=1),
        va_pickup.reshape(Bn, -1, _H, _W).any(axis=1),
        va_dig.any(axis=1),
        va_self_destruct.any(axis=1),
        va_recharge.any(axis=1),
        va_do_nothing], axis=1)
    fm = va_factory_act.any(axis=1)
    um = uatv.any(axis=1)
    crit = _conv1x1(combined, c_W, c_b)[:, 0]
    ids = location_feature.astype(jnp.int32)
    boff = (jnp.arange(Bn, dtype=jnp.int32) % 32) * 1024

    def pack_f(x):
        xt = x.transpose(1, 2, 0).reshape(_H * _W, Bn)
        return jnp.concatenate([xt, jnp.zeros((_H * _W, 128 - Bn), xt.dtype)], axis=1)

    binsu = pack_f(ids[:, 1] + boff[:, None, None])
    binsf = pack_f(ids[:, 0] + boff[:, None, None])
    cvu = pack_f(jnp.where(um, crit, 0.0))
    cvf = pack_f(jnp.where(fm, crit, 0.0))
    out = sc_scatter_v3(binsu, binsf, cvu, cvf)
    return out.reshape(Bn, 1024)[:, :_MAX_GROUP]


# speedup vs baseline: 3.9802x; 1.3684x over previous
import functools

import jax
import jax.numpy as jnp
from jax import lax
from jax.experimental import pallas as pl
from jax.experimental.pallas import tpu as pltpu
from jax.experimental.pallas import tpu_sc as plsc

_B, _H, _W = 64, 48, 48
_MAX_GROUP = 1000
_PADG = 1024
_NROW = _H * _W
_NC, _NS = 2, 16
_RPT = _NROW // _NS
_HALF = 32
_ACC = _HALF * _PADG
_PD = 2
_PH = _H + 2 * _PD


def _leaky(x):
    return jnp.where(x >= 0, x, 0.01 * x)


def _mask_body(fact_r, move_r, transfer_r, pickup_r, dig_r, sd_r, rech_r,
               dn_r, um_ref, fm_ref):
    def orall(ref):
        acc = ref[0]
        for k in range(1, ref.shape[0]):
            acc = jnp.bitwise_or(acc, ref[k])
        return acc

    fm_ref[...] = orall(fact_r)
    um = orall(move_r)
    for r in (transfer_r, pickup_r, dig_r, sd_r, rech_r):
        um = jnp.bitwise_or(um, orall(r))
    um_ref[...] = jnp.bitwise_or(um, dn_r[0])


def _tc_masks(fact, move, transfer, pickup, dig, sd, rech, dn):
    def spec(nch):
        return pl.BlockSpec((nch, 8, _W, _B), lambda h: (0, h, 0, 0))

    out_spec = pl.BlockSpec((8, _W, _B), lambda h: (h, 0, 0))
    return pl.pallas_call(
        _mask_body,
        grid=(_H // 8,),
        in_specs=[spec(4), spec(10), spec(50), spec(10), spec(2), spec(2),
                  spec(2), spec(1)],
        out_specs=[out_spec] * 2,
        out_shape=[jax.ShapeDtypeStruct((_H, _W, _B), jnp.uint8)] * 2,
    )(fact, move, transfer, pickup, dig, sd, rech, dn)


def _dense_body(gf, map_r, fac_r, unit_r, loc_r, um_r, fm_r,
                g_W, g_b, f_W, f_b, u_W, u_b, m_W, m_b, ld_W, ld_b, c_W, c_b,
                cvu_o, cvf_o, idsu_o, idsf_o, scr_me, scr_q, scr_t):
    scr_me[...] = jnp.zeros((2, _PH, _PH, _B), jnp.float32)
    scr_q[...] = jnp.zeros((2, _PH, _PH, _B), jnp.float32)
    scr_t[...] = jnp.zeros((_PH, _PH, _B), jnp.float32)

    inner = (pl.ds(_PD, _H), pl.ds(_PD, _W))

    for o in range(2):
        acc = None
        for cix in range(6):
            v = map_r[cix] * m_W[o, cix]
            acc = v if acc is None else acc + v
        scr_me[o, inner[0], inner[1], :] = _leaky(acc + m_b[o])

    for o in range(2):
        acc = None
        for dy in (-1, 0, 1):
            for dx in (-1, 0, 1):
                v = scr_me[o, pl.ds(_PD + dy, _H), pl.ds(_PD + dx, _W), :]
                acc = v if acc is None else acc + v
        scr_q[o, inner[0], inner[1], :] = acc * (1.0 / 9.0)

    def conv_row(h, _):
        z = [None] * 8
        for i in range(2):
            for dy in range(-2, 3):
                for dx in range(-2, 3):
                    s = scr_q[i, pl.ds(h + _PD + dy, 1), pl.ds(_PD + dx, _W), :]
                    for o in range(8):
                        w = ld_W[o, i * 25 + (dy + 2) * 5 + (dx + 2)]
                        z[o] = s * w if z[o] is None else z[o] + s * w
        t = None
        for o in range(8):
            u = _leaky(z[o] + ld_b[o])
            v = c_W[0, 8 + o] * u
            t = v if t is None else t + v
        scr_t[pl.ds(h + _PD, 1), pl.ds(_PD, _W), :] = t
        return 0

    lax.fori_loop(0, _H, conv_row, 0)

    acc = None
    for dy in (-1, 0, 1):
        for dx in (-1, 0, 1):
            v = scr_t[pl.ds(_PD + dy, _H), pl.ds(_PD + dx, _W), :]
            acc = v if acc is None else acc + v
    crit = acc * (1.0 / 9.0)

    g0 = gf[0:1, :]
    g1 = gf[1:2, :]
    sg = c_b[0]
    for o in range(2):
        ge = _leaky(g_W[o, 0] * g0 + g_W[o, 1] * g1 + g_b[o])
        sg = sg + c_W[0, o] * ge
    crit = crit + sg.reshape(1, 1, _B)

    for (ref, wm, bm, nch, base) in ((fac_r, f_W, f_b, 6, 2),
                                     (unit_r, u_W, u_b, 4, 4)):
        for o in range(2):
            acc = None
            for cix in range(nch):
                v = ref[cix] * wm[o, cix]
                acc = v if acc is None else acc + v
            crit = crit + c_W[0, base + o] * _leaky(acc + bm[o])
    for o in range(2):
        crit = crit + c_W[0, 6 + o] * scr_me[o, inner[0], inner[1], :]

    cvu = jnp.where(um_r[...] != 0, crit, 0.0)
    cvf = jnp.where(fm_r[...] != 0, crit, 0.0)
    bl = lax.broadcasted_iota(jnp.int32, (_H, _W, _B), 2)
    boff = (bl % _HALF) * _PADG
    binu = boff + loc_r[1].astype(jnp.int32)
    binf = boff + loc_r[0].astype(jnp.int32)
    zf = jnp.zeros((_H, _W, 128 - _B), jnp.float32)
    zi = jnp.zeros((_H, _W, 128 - _B), jnp.int32)
    cvu_o[...] = jnp.concatenate([cvu, zf], axis=2)
    cvf_o[...] = jnp.concatenate([cvf, zf], axis=2)
    idsu_o[...] = jnp.concatenate([binu, zi], axis=2)
    idsf_o[...] = jnp.concatenate([binf, zi], axis=2)


def _tc_dense(gf, map_f, fac_f, unit_f, loc_f, um, fm,
              g_W, g_b, f_W, f_b, u_W, u_b, m_W, m_b, ld_Wr, ld_b, c_W, c_b):
    vmem = pl.BlockSpec(memory_space=pltpu.VMEM)
    smem = pl.BlockSpec(memory_space=pltpu.SMEM)
    return pl.pallas_call(
        _dense_body,
        in_specs=[vmem] * 7 + [smem] * 12,
        out_specs=[vmem] * 4,
        out_shape=[jax.ShapeDtypeStruct((_H, _W, 128), jnp.float32)] * 2
        + [jax.ShapeDtypeStruct((_H, _W, 128), jnp.int32)] * 2,
        scratch_shapes=[pltpu.VMEM((2, _PH, _PH, _B), jnp.float32),
                        pltpu.VMEM((2, _PH, _PH, _B), jnp.float32),
                        pltpu.VMEM((_PH, _PH, _B), jnp.float32)],
    )(gf, map_f, fac_f, unit_f, loc_f, um, fm,
      g_W, g_b, f_W, f_b, u_W, u_b, m_W, m_b, ld_Wr, ld_b, c_W, c_b)


def _sc_scatter(ids_u, ids_f, cv_u, cv_f):
    mesh = plsc.VectorSubcoreMesh(core_axis_name="c", subcore_axis_name="s")

    @functools.partial(
        pl.kernel,
        out_type=jax.ShapeDtypeStruct((_B * _PADG,), jnp.float32),
        mesh=mesh,
        scratch_types=[
            pltpu.VMEM((_RPT, 128), jnp.int32),
            pltpu.VMEM((_RPT, 128), jnp.float32),
            pltpu.VMEM((_RPT, _HALF), jnp.int32),
            pltpu.VMEM((_RPT, _HALF), jnp.float32),
            pltpu.VMEM((_RPT, _HALF), jnp.int32),
            pltpu.VMEM((_RPT, _HALF), jnp.float32),
            pltpu.VMEM((2048,), jnp.float32),
            pltpu.VMEM_SHARED((_ACC,), jnp.float32),
            pltpu.SemaphoreType.DMA,
        ],
        compiler_params=pltpu.CompilerParams(needs_layout_passes=False),
    )
    def run(idsu_hbm, idsf_hbm, cvu_hbm, cvf_hbm, out_hbm,
            fids, fcv, cidsu, ccvu, cidsf, ccvf, zbuf, acc_sh, sem):
        c = lax.axis_index("c")
        s = lax.axis_index("s")

        zeros16 = jnp.zeros((16,), jnp.float32)

        def zb(i, _):
            zbuf[pl.ds(pl.multiple_of(i * 16, 16), 16)] = zeros16
            return 0

        lax.fori_loop(0, 2048 // 16, zb, 0)
        pltpu.sync_copy(zbuf, acc_sh.at[pl.ds(s * 2048, 2048)])

        rows = pl.ds(s * _RPT, _RPT)
        lane0 = pl.multiple_of(c * _HALF, _HALF)

        def stage(ids_hbm, cv_hbm, cids, ccv):
            pltpu.sync_copy(ids_hbm.at[rows], fids)
            pltpu.sync_copy(cv_hbm.at[rows], fcv)

            def compact(j, _):
                for t in range(2):
                    src = pl.ds(lane0 + t * 16, 16)
                    dst = pl.ds(t * 16, 16)
                    cids[j, dst] = fids[j, src]
                    ccv[j, dst] = fcv[j, src]
                return 0

            lax.fori_loop(0, _RPT, compact, 0)

        stage(idsu_hbm, cvu_hbm, cidsu, ccvu)
        stage(idsf_hbm, cvf_hbm, cidsf, ccvf)

        plsc.subcore_barrier()

        def scatter_rows(cids, ccv):
            def chunk(i, _):
                base = pl.multiple_of(i * 8, 8)
                ds_ = [pltpu.async_copy(ccv.at[base + jj],
                                        acc_sh.at[cids.at[base + jj]],
                                        sem, add=True)
                       for jj in range(8)]
                for d in ds_:
                    d.wait()
                return 0

            lax.fori_loop(0, _RPT // 8, chunk, 0)

        scatter_rows(cidsu, ccvu)
        scatter_rows(cidsf, ccvf)

        plsc.subcore_barrier()

        pltpu.sync_copy(acc_sh.at[pl.ds(s * 2048, 2048)],
                        out_hbm.at[pl.ds(c * _ACC + s * 2048, 2048)])

    return run(ids_u, ids_f, cv_u, cv_f)


def _bm(x):
    perm = tuple(range(1, x.ndim)) + (0,)
    return x.transpose(perm)


def kernel(global_feature, map_feature, factory_feature, unit_feature,
           location_feature, va_factory_act, va_move, va_transfer, va_pickup,
           va_dig, va_self_destruct, va_recharge, va_do_nothing,
           g_W, g_b, f_W, f_b, u_W, u_b, m_W, m_b, ld_W, ld_b, c_W, c_b):
    Bn = global_feature.shape[0]

    def u8(x):
        return _bm(x.astype(jnp.uint8)).reshape(-1, _H, _W, Bn)

    um, fm = _tc_masks(u8(va_factory_act), u8(va_move), u8(va_transfer),
                       u8(va_pickup), u8(va_dig), u8(va_self_destruct),
                       u8(va_recharge), u8(va_do_nothing[:, None]))
    cv_u, cv_f, ids_u, ids_f = _tc_dense(
        global_feature.transpose(1, 0), _bm(map_feature),
        _bm(factory_feature), _bm(unit_feature), _bm(location_feature),
        um, fm,
        g_W, g_b, f_W, f_b, u_W, u_b, m_W, m_b,
        ld_W.reshape(8, 50), ld_b, c_W, c_b)
    out = _sc_scatter(ids_u.reshape(_NROW, 128), ids_f.reshape(_NROW, 128),
                      cv_u.reshape(_NROW, 128), cv_f.reshape(_NROW, 128))
    return out.reshape(Bn, _PADG)[:, :_MAX_GROUP]


# speedup vs baseline: 4.6628x; 1.1715x over previous
import functools

import jax
import jax.numpy as jnp
from jax import lax
from jax.experimental import pallas as pl
from jax.experimental.pallas import tpu as pltpu
from jax.experimental.pallas import tpu_sc as plsc

_B, _H, _W = 64, 48, 48
_MAX_GROUP = 1000
_PADG = 1024
_NROW = _H * _W
_NC, _NS = 2, 16
_RPT = _NROW // _NS
_HALF = 32
_ACC = _HALF * _PADG
_PD = 2
_PH = _H + 2 * _PD


def _leaky(x):
    return jnp.where(x >= 0, x, 0.01 * x)


def _mask_body(fact_r, move_r, transfer_r, pickup_r, dig_r, sd_r, rech_r,
               dn_r, um_ref, fm_ref):
    def orall(ref):
        acc = ref[0]
        for k in range(1, ref.shape[0]):
            acc = jnp.bitwise_or(acc, ref[k])
        return acc

    fm_ref[...] = orall(fact_r)
    um = orall(move_r)
    for r in (transfer_r, pickup_r, dig_r, sd_r, rech_r):
        um = jnp.bitwise_or(um, orall(r))
    um_ref[...] = jnp.bitwise_or(um, dn_r[0])


def _tc_masks(fact, move, transfer, pickup, dig, sd, rech, dn):
    def spec(nch):
        return pl.BlockSpec((nch, 8, _W, _B), lambda h: (0, h, 0, 0))

    out_spec = pl.BlockSpec((8, _W, _B), lambda h: (h, 0, 0))
    return pl.pallas_call(
        _mask_body,
        grid=(_H // 8,),
        in_specs=[spec(4), spec(10), spec(50), spec(10), spec(2), spec(2),
                  spec(2), spec(1)],
        out_specs=[out_spec] * 2,
        out_shape=[jax.ShapeDtypeStruct((_H, _W, _B), jnp.uint8)] * 2,
    )(fact, move, transfer, pickup, dig, sd, rech, dn)


def _dense_body(gf, map_r, fac_r, unit_r, loc_r, um_r, fm_r,
                g_W, g_b, f_W, f_b, u_W, u_b, m_W, m_b, ld_W, ld_b, c_W, c_b,
                cvu_o, cvf_o, idsu_o, idsf_o, scr_me, scr_pq, scr_t):
    zrow = jnp.zeros((2, 1, 50, _B), jnp.float32)
    zcol = jnp.zeros((2, 50, 1, _B), jnp.float32)
    for r in (1, 50):
        scr_me[:, pl.ds(r, 1), pl.ds(1, 50), :] = zrow
        scr_t[pl.ds(r, 1), pl.ds(1, 50), :] = zrow[0]
    for cix in (1, 50):
        scr_me[:, pl.ds(1, 50), pl.ds(cix, 1), :] = zcol
        scr_t[pl.ds(1, 50), pl.ds(cix, 1), :] = zcol[0]
    scr_pq[...] = jnp.zeros((2, _PH, _PH, 128), jnp.float32)

    inner = (pl.ds(_PD, _H), pl.ds(_PD, _W))

    for o in range(2):
        acc = None
        for cix in range(6):
            v = map_r[cix] * m_W[o, cix]
            acc = v if acc is None else acc + v
        scr_me[o, inner[0], inner[1], :] = _leaky(acc + m_b[o])

    for o in range(2):
        acc = None
        for dy in (-1, 0, 1):
            for dx in (-1, 0, 1):
                v = scr_me[o, pl.ds(_PD + dy, _H), pl.ds(_PD + dx, _W), :]
                acc = v if acc is None else acc + v
        q = acc * (1.0 / 9.0)
        scr_pq[o, pl.ds(_PD, _H), pl.ds(_PD, _W), pl.ds(0, _B)] = q
        scr_pq[o, pl.ds(_PD - 1, _H), pl.ds(_PD, _W), pl.ds(_B, _B)] = q

    def conv_pair(hp, _):
        h = hp * 2
        z = [None] * 8
        for i in range(2):
            for dy in range(-2, 3):
                for dx in range(-2, 3):
                    s = scr_pq[i, pl.ds(h + _PD + dy, 1),
                               pl.ds(_PD + dx, _W), :]
                    for o in range(8):
                        w = ld_W[o, i * 25 + (dy + 2) * 5 + (dx + 2)]
                        z[o] = s * w if z[o] is None else z[o] + s * w
        t = None
        for o in range(8):
            u = _leaky(z[o] + ld_b[o])
            v = c_W[0, 8 + o] * u
            t = v if t is None else t + v
        scr_t[pl.ds(h + _PD, 1), pl.ds(_PD, _W), :] = t[:, :, :_B]
        scr_t[pl.ds(h + _PD + 1, 1), pl.ds(_PD, _W), :] = t[:, :, _B:]
        return 0

    lax.fori_loop(0, _H // 2, conv_pair, 0)

    acc = None
    for dy in (-1, 0, 1):
        for dx in (-1, 0, 1):
            v = scr_t[pl.ds(_PD + dy, _H), pl.ds(_PD + dx, _W), :]
            acc = v if acc is None else acc + v
    crit = acc * (1.0 / 9.0)

    g0 = gf[0:1, :]
    g1 = gf[1:2, :]
    sg = c_b[0]
    for o in range(2):
        ge = _leaky(g_W[o, 0] * g0 + g_W[o, 1] * g1 + g_b[o])
        sg = sg + c_W[0, o] * ge
    crit = crit + sg.reshape(1, 1, _B)

    for (ref, wm, bm, nch, base) in ((fac_r, f_W, f_b, 6, 2),
                                     (unit_r, u_W, u_b, 4, 4)):
        for o in range(2):
            acc = None
            for cix in range(nch):
                v = ref[cix] * wm[o, cix]
                acc = v if acc is None else acc + v
            crit = crit + c_W[0, base + o] * _leaky(acc + bm[o])
    for o in range(2):
        crit = crit + c_W[0, 6 + o] * scr_me[o, inner[0], inner[1], :]

    cvu = jnp.where(um_r[...] != 0, crit, 0.0)
    cvf = jnp.where(fm_r[...] != 0, crit, 0.0)
    bl = lax.broadcasted_iota(jnp.int32, (_H, _W, _B), 2)
    boff = (bl % _HALF) * _PADG
    binu = boff + loc_r[1].astype(jnp.int32)
    binf = boff + loc_r[0].astype(jnp.int32)
    zf = jnp.zeros((_H, _W, 128 - _B), jnp.float32)
    zi = jnp.zeros((_H, _W, 128 - _B), jnp.int32)
    cvu_o[...] = jnp.concatenate([cvu, zf], axis=2)
    cvf_o[...] = jnp.concatenate([cvf, zf], axis=2)
    idsu_o[...] = jnp.concatenate([binu, zi], axis=2)
    idsf_o[...] = jnp.concatenate([binf, zi], axis=2)


def _tc_dense(gf, map_f, fac_f, unit_f, loc_f, um, fm,
              g_W, g_b, f_W, f_b, u_W, u_b, m_W, m_b, ld_Wr, ld_b, c_W, c_b):
    vmem = pl.BlockSpec(memory_space=pltpu.VMEM)
    smem = pl.BlockSpec(memory_space=pltpu.SMEM)
    return pl.pallas_call(
        _dense_body,
        in_specs=[vmem] * 7 + [smem] * 12,
        out_specs=[vmem] * 4,
        out_shape=[jax.ShapeDtypeStruct((_H, _W, 128), jnp.float32)] * 2
        + [jax.ShapeDtypeStruct((_H, _W, 128), jnp.int32)] * 2,
        scratch_shapes=[pltpu.VMEM((2, _PH, _PH, _B), jnp.float32),
                        pltpu.VMEM((2, _PH, _PH, 128), jnp.float32),
                        pltpu.VMEM((_PH, _PH, _B), jnp.float32)],
    )(gf, map_f, fac_f, unit_f, loc_f, um, fm,
      g_W, g_b, f_W, f_b, u_W, u_b, m_W, m_b, ld_Wr, ld_b, c_W, c_b)


def _sc_scatter(ids_u, ids_f, cv_u, cv_f):
    mesh = plsc.VectorSubcoreMesh(core_axis_name="c", subcore_axis_name="s")

    @functools.partial(
        pl.kernel,
        out_type=jax.ShapeDtypeStruct((_B * _PADG,), jnp.float32),
        mesh=mesh,
        scratch_types=[
            pltpu.VMEM((_RPT, 128), jnp.int32),
            pltpu.VMEM((_RPT, 128), jnp.float32),
            pltpu.VMEM((_RPT, _HALF), jnp.int32),
            pltpu.VMEM((_RPT, _HALF), jnp.float32),
            pltpu.VMEM((_RPT, _HALF), jnp.int32),
            pltpu.VMEM((_RPT, _HALF), jnp.float32),
            pltpu.VMEM((2048,), jnp.float32),
            pltpu.VMEM_SHARED((_ACC,), jnp.float32),
            pltpu.SemaphoreType.DMA,
        ],
        compiler_params=pltpu.CompilerParams(needs_layout_passes=False),
    )
    def run(idsu_hbm, idsf_hbm, cvu_hbm, cvf_hbm, out_hbm,
            fids, fcv, cidsu, ccvu, cidsf, ccvf, zbuf, acc_sh, sem):
        c = lax.axis_index("c")
        s = lax.axis_index("s")

        zeros16 = jnp.zeros((16,), jnp.float32)

        def zb(i, _):
            zbuf[pl.ds(pl.multiple_of(i * 16, 16), 16)] = zeros16
            return 0

        lax.fori_loop(0, 2048 // 16, zb, 0)
        pltpu.sync_copy(zbuf, acc_sh.at[pl.ds(s * 2048, 2048)])

        rows = pl.ds(s * _RPT, _RPT)
        lane0 = pl.multiple_of(c * _HALF, _HALF)

        def stage(ids_hbm, cv_hbm, cids, ccv):
            pltpu.sync_copy(ids_hbm.at[rows], fids)
            pltpu.sync_copy(cv_hbm.at[rows], fcv)

            def compact(j, _):
                for t in range(2):
                    src = pl.ds(lane0 + t * 16, 16)
                    dst = pl.ds(t * 16, 16)
                    cids[j, dst] = fids[j, src]
                    ccv[j, dst] = fcv[j, src]
                return 0

            lax.fori_loop(0, _RPT, compact, 0)

        stage(idsu_hbm, cvu_hbm, cidsu, ccvu)
        stage(idsf_hbm, cvf_hbm, cidsf, ccvf)

        plsc.subcore_barrier()

        def scatter_rows(cids, ccv):
            def chunk(i, _):
                base = pl.multiple_of(i * 8, 8)
                ds_ = [pltpu.async_copy(ccv.at[base + jj],
                                        acc_sh.at[cids.at[base + jj]],
                                        sem, add=True)
                       for jj in range(8)]
                for d in ds_:
                    d.wait()
                return 0

            lax.fori_loop(0, _RPT // 8, chunk, 0)

        scatter_rows(cidsu, ccvu)
        scatter_rows(cidsf, ccvf)

        plsc.subcore_barrier()

        pltpu.sync_copy(acc_sh.at[pl.ds(s * 2048, 2048)],
                        out_hbm.at[pl.ds(c * _ACC + s * 2048, 2048)])

    return run(ids_u, ids_f, cv_u, cv_f)


def _bm(x):
    perm = tuple(range(1, x.ndim)) + (0,)
    return x.transpose(perm)


def kernel(global_feature, map_feature, factory_feature, unit_feature,
           location_feature, va_factory_act, va_move, va_transfer, va_pickup,
           va_dig, va_self_destruct, va_recharge, va_do_nothing,
           g_W, g_b, f_W, f_b, u_W, u_b, m_W, m_b, ld_W, ld_b, c_W, c_b):
    Bn = global_feature.shape[0]

    def u8(x):
        return _bm(x.astype(jnp.uint8)).reshape(-1, _H, _W, Bn)

    um, fm = _tc_masks(u8(va_factory_act), u8(va_move), u8(va_transfer),
                       u8(va_pickup), u8(va_dig), u8(va_self_destruct),
                       u8(va_recharge), u8(va_do_nothing[:, None]))
    cv_u, cv_f, ids_u, ids_f = _tc_dense(
        global_feature.transpose(1, 0), _bm(map_feature),
        _bm(factory_feature), _bm(unit_feature), _bm(location_feature),
        um, fm,
        g_W, g_b, f_W, f_b, u_W, u_b, m_W, m_b,
        ld_W.reshape(8, 50), ld_b, c_W, c_b)
    out = _sc_scatter(ids_u.reshape(_NROW, 128), ids_f.reshape(_NROW, 128),
                      cv_u.reshape(_NROW, 128), cv_f.reshape(_NROW, 128))
    return out.reshape(Bn, _PADG)[:, :_MAX_GROUP]


# speedup vs baseline: 5.0069x; 1.0738x over previous
import functools

import jax
import jax.numpy as jnp
from jax import lax
from jax.experimental import pallas as pl
from jax.experimental.pallas import tpu as pltpu
from jax.experimental.pallas import tpu_sc as plsc

_B, _H, _W = 64, 48, 48
_MAX_GROUP = 1000
_PADG = 1024
_NROW = _H * _W
_NC, _NS = 2, 16
_RPT = _NROW // _NS
_HALF = 32
_ACC = _HALF * _PADG
_PD = 2
_PH = _H + 2 * _PD


def _leaky(x):
    return jnp.where(x >= 0, x, 0.01 * x)


def _mask_body(fact_r, move_r, transfer_r, pickup_r, dig_r, sd_r, rech_r,
               dn_r, um_ref, fm_ref):
    def orall(ref):
        acc = ref[0]
        for k in range(1, ref.shape[0]):
            acc = jnp.bitwise_or(acc, ref[k])
        return acc

    fm_ref[...] = orall(fact_r)
    um = orall(move_r)
    for r in (transfer_r, pickup_r, dig_r, sd_r, rech_r):
        um = jnp.bitwise_or(um, orall(r))
    um_ref[...] = jnp.bitwise_or(um, dn_r[0])


def _tc_masks(fact, move, transfer, pickup, dig, sd, rech, dn):
    def spec(nch):
        return pl.BlockSpec((nch, 8, _W, _B), lambda h: (0, h, 0, 0))

    out_spec = pl.BlockSpec((8, _W, _B), lambda h: (h, 0, 0))
    return pl.pallas_call(
        _mask_body,
        grid=(_H // 8,),
        in_specs=[spec(4), spec(10), spec(50), spec(10), spec(2), spec(2),
                  spec(2), spec(1)],
        out_specs=[out_spec] * 2,
        out_shape=[jax.ShapeDtypeStruct((_H, _W, _B), jnp.uint8)] * 2,
    )(fact, move, transfer, pickup, dig, sd, rech, dn)


def _dense_body(gf, map_r, fac_r, unit_r, loc_r, um_r, fm_r,
                g_W, g_b, f_W, f_b, u_W, u_b, m_W, m_b, ld_W, ld_b, c_W, c_b,
                cvu_o, cvf_o, idsu_o, idsf_o, scr_me, scr_pq, scr_t):
    zrow = jnp.zeros((2, 1, 50, _B), jnp.float32)
    zcol = jnp.zeros((2, 50, 1, _B), jnp.float32)
    for r in (1, 50):
        scr_me[:, pl.ds(r, 1), pl.ds(1, 50), :] = zrow
        scr_t[pl.ds(r, 1), pl.ds(1, 50), :] = zrow[0]
    for cix in (1, 50):
        scr_me[:, pl.ds(1, 50), pl.ds(cix, 1), :] = zcol
        scr_t[pl.ds(1, 50), pl.ds(cix, 1), :] = zcol[0]
    scr_pq[...] = jnp.zeros((2, _PH, _PH, 128), jnp.float32)

    inner = (pl.ds(_PD, _H), pl.ds(_PD, _W))

    for o in range(2):
        acc = None
        for cix in range(6):
            v = map_r[cix] * m_W[o, cix]
            acc = v if acc is None else acc + v
        scr_me[o, inner[0], inner[1], :] = _leaky(acc + m_b[o])

    for o in range(2):
        acc = None
        for dy in (-1, 0, 1):
            for dx in (-1, 0, 1):
                v = scr_me[o, pl.ds(_PD + dy, _H), pl.ds(_PD + dx, _W), :]
                acc = v if acc is None else acc + v
        q = acc * (1.0 / 9.0)
        scr_pq[o, pl.ds(_PD, _H), pl.ds(_PD, _W), pl.ds(0, _B)] = q
        scr_pq[o, pl.ds(_PD - 1, _H), pl.ds(_PD, _W), pl.ds(_B, _B)] = q

    def conv_pair(hp, _):
        h = hp * 2
        z = [None] * 8
        for i in range(2):
            for dy in range(-2, 3):
                for dx in range(-2, 3):
                    s = scr_pq[i, pl.ds(h + _PD + dy, 1),
                               pl.ds(_PD + dx, _W), :]
                    for o in range(8):
                        w = ld_W[o, i * 25 + (dy + 2) * 5 + (dx + 2)]
                        z[o] = s * w if z[o] is None else z[o] + s * w
        t = None
        for o in range(8):
            u = _leaky(z[o] + ld_b[o])
            v = c_W[0, 8 + o] * u
            t = v if t is None else t + v
        scr_t[pl.ds(h + _PD, 1), pl.ds(_PD, _W), :] = t[:, :, :_B]
        scr_t[pl.ds(h + _PD + 1, 1), pl.ds(_PD, _W), :] = t[:, :, _B:]
        return 0

    lax.fori_loop(0, _H // 2, conv_pair, 0)

    acc = None
    for dy in (-1, 0, 1):
        for dx in (-1, 0, 1):
            v = scr_t[pl.ds(_PD + dy, _H), pl.ds(_PD + dx, _W), :]
            acc = v if acc is None else acc + v
    crit = acc * (1.0 / 9.0)

    g0 = gf[0:1, :]
    g1 = gf[1:2, :]
    sg = c_b[0]
    for o in range(2):
        ge = _leaky(g_W[o, 0] * g0 + g_W[o, 1] * g1 + g_b[o])
        sg = sg + c_W[0, o] * ge
    crit = crit + sg.reshape(1, 1, _B)

    for (ref, wm, bm, nch, base) in ((fac_r, f_W, f_b, 6, 2),
                                     (unit_r, u_W, u_b, 4, 4)):
        for o in range(2):
            acc = None
            for cix in range(nch):
                v = ref[cix] * wm[o, cix]
                acc = v if acc is None else acc + v
            crit = crit + c_W[0, base + o] * _leaky(acc + bm[o])
    for o in range(2):
        crit = crit + c_W[0, 6 + o] * scr_me[o, inner[0], inner[1], :]

    cvu = jnp.where(um_r[...] != 0, crit, 0.0)
    cvf = jnp.where(fm_r[...] != 0, crit, 0.0)
    bl = lax.broadcasted_iota(jnp.int32, (_H, _W, _B), 2)
    boff = (bl % _HALF) * _PADG
    binu = boff + loc_r[1].astype(jnp.int32)
    binf = boff + loc_r[0].astype(jnp.int32)
    zf = jnp.zeros((_H, _W, 128 - _B), jnp.float32)
    zi = jnp.zeros((_H, _W, 128 - _B), jnp.int32)
    cvu_o[...] = jnp.concatenate([cvu, zf], axis=2)
    cvf_o[...] = jnp.concatenate([cvf, zf], axis=2)
    idsu_o[...] = jnp.concatenate([binu, zi], axis=2)
    idsf_o[...] = jnp.concatenate([binf, zi], axis=2)


def _tc_dense(gf, map_f, fac_f, unit_f, loc_f, um, fm,
              g_W, g_b, f_W, f_b, u_W, u_b, m_W, m_b, ld_Wr, ld_b, c_W, c_b):
    vmem = pl.BlockSpec(memory_space=pltpu.VMEM)
    smem = pl.BlockSpec(memory_space=pltpu.SMEM)
    return pl.pallas_call(
        _dense_body,
        in_specs=[vmem] * 7 + [smem] * 12,
        out_specs=[vmem] * 4,
        out_shape=[jax.ShapeDtypeStruct((_H, _W, 128), jnp.float32)] * 2
        + [jax.ShapeDtypeStruct((_H, _W, 128), jnp.int32)] * 2,
        scratch_shapes=[pltpu.VMEM((2, _PH, _PH, _B), jnp.float32),
                        pltpu.VMEM((2, _PH, _PH, 128), jnp.float32),
                        pltpu.VMEM((_PH, _PH, _B), jnp.float32)],
    )(gf, map_f, fac_f, unit_f, loc_f, um, fm,
      g_W, g_b, f_W, f_b, u_W, u_b, m_W, m_b, ld_Wr, ld_b, c_W, c_b)


def _sc_scatter(ids_u, ids_f, cv_u, cv_f):
    mesh = plsc.VectorSubcoreMesh(core_axis_name="c", subcore_axis_name="s")

    @functools.partial(
        pl.kernel,
        out_type=jax.ShapeDtypeStruct((_B * _PADG,), jnp.float32),
        mesh=mesh,
        scratch_types=[
            pltpu.VMEM((_RPT, 128), jnp.int32),
            pltpu.VMEM((_RPT, 128), jnp.float32),
            pltpu.VMEM((_RPT, 128), jnp.int32),
            pltpu.VMEM((_RPT, 128), jnp.float32),
            pltpu.VMEM((2048,), jnp.float32),
            pltpu.VMEM_SHARED((_ACC,), jnp.float32),
            pltpu.SemaphoreType.DMA,
            pltpu.SemaphoreType.DMA,
        ],
        compiler_params=pltpu.CompilerParams(needs_layout_passes=False),
    )
    def run(idsu_hbm, idsf_hbm, cvu_hbm, cvf_hbm, out_hbm,
            fidsu, fcvu, fidsf, fcvf, zbuf, acc_sh, sem, sem2):
        c = lax.axis_index("c")
        s = lax.axis_index("s")

        rows = pl.ds(s * _RPT, _RPT)
        st = [pltpu.async_copy(idsu_hbm.at[rows], fidsu, sem2),
              pltpu.async_copy(cvu_hbm.at[rows], fcvu, sem2),
              pltpu.async_copy(idsf_hbm.at[rows], fidsf, sem2),
              pltpu.async_copy(cvf_hbm.at[rows], fcvf, sem2)]

        zeros16 = jnp.zeros((16,), jnp.float32)

        def zb(i, _):
            zbuf[pl.ds(pl.multiple_of(i * 16, 16), 16)] = zeros16
            return 0

        lax.fori_loop(0, 2048 // 16, zb, 0)
        pltpu.sync_copy(zbuf, acc_sh.at[pl.ds(s * 2048, 2048)])
        for d in st:
            d.wait()

        plsc.subcore_barrier()

        lane0 = pl.multiple_of(c * _HALF, _HALF)
        lanes = pl.ds(lane0, _HALF)

        def scatter_rows(fids, fcv):
            def chunk(i, _):
                base = pl.multiple_of(i * 8, 8)
                ds_ = [pltpu.async_copy(fcv.at[base + jj, lanes],
                                        acc_sh.at[fids.at[base + jj, lanes]],
                                        sem, add=True)
                       for jj in range(8)]
                for d in ds_:
                    d.wait()
                return 0

            lax.fori_loop(0, _RPT // 8, chunk, 0)

        scatter_rows(fidsu, fcvu)
        scatter_rows(fidsf, fcvf)

        plsc.subcore_barrier()

        pltpu.sync_copy(acc_sh.at[pl.ds(s * 2048, 2048)],
                        out_hbm.at[pl.ds(c * _ACC + s * 2048, 2048)])

    return run(ids_u, ids_f, cv_u, cv_f)


def _bm(x):
    perm = tuple(range(1, x.ndim)) + (0,)
    return x.transpose(perm)


def kernel(global_feature, map_feature, factory_feature, unit_feature,
           location_feature, va_factory_act, va_move, va_transfer, va_pickup,
           va_dig, va_self_destruct, va_recharge, va_do_nothing,
           g_W, g_b, f_W, f_b, u_W, u_b, m_W, m_b, ld_W, ld_b, c_W, c_b):
    Bn = global_feature.shape[0]

    def u8(x):
        return _bm(x.astype(jnp.uint8)).reshape(-1, _H, _W, Bn)

    um, fm = _tc_masks(u8(va_factory_act), u8(va_move), u8(va_transfer),
                       u8(va_pickup), u8(va_dig), u8(va_self_destruct),
                       u8(va_recharge), u8(va_do_nothing[:, None]))
    cv_u, cv_f, ids_u, ids_f = _tc_dense(
        global_feature.transpose(1, 0), _bm(map_feature),
        _bm(factory_feature), _bm(unit_feature), _bm(location_feature),
        um, fm,
        g_W, g_b, f_W, f_b, u_W, u_b, m_W, m_b,
        ld_W.reshape(8, 50), ld_b, c_W, c_b)
    out = _sc_scatter(ids_u.reshape(_NROW, 128), ids_f.reshape(_NROW, 128),
                      cv_u.reshape(_NROW, 128), cv_f.reshape(_NROW, 128))
    return out.reshape(Bn, _PADG)[:, :_MAX_GROUP]
